# baseline (speedup 1.0000x reference)
import functools

import jax
import jax.numpy as jnp
from jax import lax
from jax.experimental import pallas as pl
from jax.experimental.pallas import tpu as pltpu

D_MODEL = 768
N_EXPERTS = 8
TOP_K = 2
D_FF = 4 * D_MODEL
T = 2048

_NEG_INF = float("-inf")


def _router_body(x_ref, w_ref, b_ref, u_ref, gate_ref):
    z = jax.lax.dot_general(
        x_ref[...], w_ref[...], (((1,), (0,)), ((), ())),
        preferred_element_type=jnp.float32,
        precision=jax.lax.Precision.DEFAULT,
    ) + b_ref[...]
    logits = z[:, 0:N_EXPERTS]
    nlog = z[:, N_EXPERTS:2 * N_EXPERTS]
    sp = jnp.maximum(nlog, 0.0) + jnp.log1p(jnp.exp(-jnp.abs(nlog)))
    noisy = logits + sp * u_ref[...]

    lane = lax.broadcasted_iota(jnp.int32, (T, N_EXPERTS), 1)
    m1 = jnp.max(noisy, axis=1, keepdims=True)
    i1 = jnp.min(jnp.where(noisy == m1, lane, N_EXPERTS), axis=1,
                 keepdims=True)
    masked = jnp.where(lane == i1, _NEG_INF, noisy)
    m2 = jnp.max(masked, axis=1, keepdims=True)
    i2 = jnp.min(jnp.where(masked == m2, lane, N_EXPERTS), axis=1,
                 keepdims=True)
    e2 = jnp.exp(m2 - m1)
    denom = 1.0 + e2
    g1 = 1.0 / denom
    g2 = e2 / denom
    gate_ref[...] = jnp.where(lane == i1, g1,
                              jnp.where(lane == i2, g2, 0.0))


def _ffn_body(x_ref, w1_ref, b1_ref, w2_ref, b2_ref, g_ref, out_ref):
    e = pl.program_id(0)

    @pl.when(e == 0)
    def _init():
        out_ref[...] = jnp.zeros_like(out_ref)

    w1 = w1_ref[0]
    w2 = w2_ref[0]
    b1 = b1_ref[0]
    b2 = b2_ref[0]
    chunk = 512
    for c in range(T // chunk):
        sl = pl.ds(c * chunk, chunk)
        xb = x_ref[sl, :].astype(jnp.bfloat16)
        h = jax.lax.dot_general(
            xb, w1, (((1,), (0,)), ((), ())),
            preferred_element_type=jnp.float32) + b1
        h = jnp.maximum(h, 0.0).astype(jnp.bfloat16)
        o = jax.lax.dot_general(
            h, w2, (((1,), (0,)), ((), ())),
            preferred_element_type=jnp.float32) + b2
        gblk = g_ref[sl, :]
        lane8 = lax.broadcasted_iota(jnp.int32, (chunk, N_EXPERTS), 1)
        g = jnp.sum(jnp.where(lane8 == e, gblk, 0.0), axis=1, keepdims=True)
        out_ref[sl, :] += o * g


def kernel(x, Wr, br, Wn, bn, W1, b1, W2, b2):
    xf = x.reshape(T, D_MODEL)
    w_pack = jnp.zeros((D_MODEL, 128), jnp.float32)
    w_pack = w_pack.at[:, 0:N_EXPERTS].set(Wr)
    w_pack = w_pack.at[:, N_EXPERTS:2 * N_EXPERTS].set(Wn)
    b_pack = jnp.zeros((1, 128), jnp.float32)
    b_pack = b_pack.at[0, 0:N_EXPERTS].set(br)
    b_pack = b_pack.at[0, N_EXPERTS:2 * N_EXPERTS].set(bn)
    u = jax.random.uniform(jax.random.key(42), (1, T, N_EXPERTS),
                           dtype=jnp.float32).reshape(T, N_EXPERTS)

    gating = pl.pallas_call(
        _router_body,
        out_shape=jax.ShapeDtypeStruct((T, N_EXPERTS), jnp.float32),
    )(xf, w_pack, b_pack, u)

    w1b = W1.astype(jnp.bfloat16)
    w2b = W2.astype(jnp.bfloat16)

    out = pl.pallas_call(
        _ffn_body,
        grid=(N_EXPERTS,),
        in_specs=[
            pl.BlockSpec((T, D_MODEL), lambda e: (0, 0)),
            pl.BlockSpec((1, D_MODEL, D_FF), lambda e: (e, 0, 0)),
            pl.BlockSpec((1, 1, D_FF), lambda e: (e, 0, 0)),
            pl.BlockSpec((1, D_FF, D_MODEL), lambda e: (e, 0, 0)),
            pl.BlockSpec((1, 1, D_MODEL), lambda e: (e, 0, 0)),
            pl.BlockSpec((T, N_EXPERTS), lambda e: (0, 0)),
        ],
        out_specs=pl.BlockSpec((T, D_MODEL), lambda e: (0, 0)),
        out_shape=jax.ShapeDtypeStruct((T, D_MODEL), jnp.float32),
    )(xf, w1b, b1.reshape(N_EXPERTS, 1, D_FF), w2b,
      b2.reshape(N_EXPERTS, 1, D_MODEL), gating)

    return out.reshape(x.shape)
